# baseline (speedup 1.0000x reference)
import functools

import jax
import jax.numpy as jnp
from jax import lax
from jax.experimental import pallas as pl
from jax.experimental.pallas import tpu as pltpu
from jax.experimental.pallas import tpu_sc as plsc

D = 128
NF = D // 2
B = 16384
SEQ = 4096
NB = B // SEQ
NT = SEQ // 128
NG = B // 128
NC, NS = 2, 16
NW = NC * NS
PER_W = B // NW
CHUNK = 128
NCHUNK = PER_W // CHUNK
NORM_BLK = 544


def _norm_body(w_ref, o_ref):
    w = w_ref[...]
    sq = w * w
    left = jnp.concatenate([sq[:, 1:], sq[:, :1]], axis=1)
    right = jnp.concatenate([sq[:, :1], sq[:, :-1]], axis=1)
    lane = lax.broadcasted_iota(jnp.int32, w.shape, 1)
    pair = sq + jnp.where(lane % 2 == 0, left, right)
    o_ref[...] = w / jnp.sqrt(pair)


def _normalize(w):
    n = w.shape[0]
    return pl.pallas_call(
        _norm_body,
        grid=(pl.cdiv(n, NORM_BLK),),
        in_specs=[pl.BlockSpec((NORM_BLK, D), lambda i: (i, 0))],
        out_specs=pl.BlockSpec((NORM_BLK, D), lambda i: (i, 0)),
        out_shape=jax.ShapeDtypeStruct(w.shape, w.dtype),
    )(w)


def _gather_body(table, idx, out, idx_v, rows, sem):
    wid = lax.axis_index("s") * NC + lax.axis_index("c")
    b = wid // (NT // NCHUNK)
    col0 = (wid % (NT // NCHUNK)) * PER_W
    pltpu.sync_copy(idx.at[b, pl.ds(col0, PER_W)], idx_v)
    c = pltpu.make_async_copy(table.at[idx_v], rows, sem)
    c.start()
    c.wait()
    pltpu.sync_copy(rows, out.at[wid])


def _make_gather():
    mesh = plsc.VectorSubcoreMesh(
        core_axis_name="c", subcore_axis_name="s",
        num_cores=NC, num_subcores=NS,
    )
    return pl.kernel(
        _gather_body,
        out_type=jax.ShapeDtypeStruct((NW, PER_W, D), jnp.float32),
        mesh=mesh,
        scratch_types=[
            pltpu.VMEM((PER_W,), jnp.int32),
            pltpu.VMEM((PER_W, D), jnp.float32),
            pltpu.SemaphoreType.DMA,
        ],
        compiler_params=pltpu.CompilerParams(needs_layout_passes=False),
    )


def _trans_body(x_ref, o_ref):
    parts = [x_ref[t].T.reshape(NF, 2, CHUNK) for t in range(NT // 2)]
    o_ref[0] = jnp.concatenate(parts, axis=1)


def _transpose(r4):
    return pl.pallas_call(
        _trans_body,
        grid=(2 * NB,),
        in_specs=[pl.BlockSpec((NT // 2, CHUNK, D), lambda i: (i, 0, 0))],
        out_specs=pl.BlockSpec((1, NF, NT, CHUNK),
                               lambda i: (i // 2, 0, i % 2, 0)),
        out_shape=jax.ShapeDtypeStruct((NB, NF, 2 * NT, CHUNK), jnp.float32),
    )(r4)


def kernel(current_position_ids, past_position_ids, pos_emb_weight):
    w = _normalize(pos_emb_weight)
    gather = _make_gather()
    r_a = gather(w, current_position_ids).reshape(NG, CHUNK, D)
    r_b = gather(w, past_position_ids).reshape(NG, CHUNK, D)

    shp = current_position_ids.shape

    def assemble(p):
        f = p.reshape(NB, NF, NT, 2, CHUNK).transpose(0, 2, 4, 1, 3)
        return f.reshape(shp[0], shp[1], NF, 2)[:, None]

    return (assemble(_transpose(r_a)), assemble(_transpose(r_b)))


# speedup vs baseline: 1.0462x; 1.0462x over previous
import functools

import jax
import jax.numpy as jnp
from jax import lax
from jax.experimental import pallas as pl
from jax.experimental.pallas import tpu as pltpu
from jax.experimental.pallas import tpu_sc as plsc

D = 128
NF = D // 2
B = 16384
SEQ = 4096
NB = B // SEQ
NT = SEQ // 128
NG = B // 128
NC, NS = 2, 16
NW = NC * NS
PER_W = B // NW
CHUNK = 128
NCHUNK = PER_W // CHUNK
NORM_BLK = 544


def _norm_body(w_ref, o_ref):
    w = w_ref[...]
    sq = w * w
    left = jnp.concatenate([sq[:, 1:], sq[:, :1]], axis=1)
    right = jnp.concatenate([sq[:, :1], sq[:, :-1]], axis=1)
    lane = lax.broadcasted_iota(jnp.int32, w.shape, 1)
    pair = sq + jnp.where(lane % 2 == 0, left, right)
    o_ref[...] = w / jnp.sqrt(pair)


def _normalize(w):
    n = w.shape[0]
    return pl.pallas_call(
        _norm_body,
        grid=(pl.cdiv(n, NORM_BLK),),
        in_specs=[pl.BlockSpec((NORM_BLK, D), lambda i: (i, 0))],
        out_specs=pl.BlockSpec((NORM_BLK, D), lambda i: (i, 0)),
        out_shape=jax.ShapeDtypeStruct(w.shape, w.dtype),
    )(w)


def _gather_body(table, idx, out, idx_v, rows, sem):
    wid = lax.axis_index("s") * NC + lax.axis_index("c")
    b = wid // (NT // NCHUNK)
    col0 = (wid % (NT // NCHUNK)) * PER_W
    pltpu.sync_copy(idx.at[b, pl.ds(col0, PER_W)], idx_v)
    c = pltpu.make_async_copy(table.at[idx_v], rows, sem)
    c.start()
    c.wait()
    pltpu.sync_copy(rows, out.at[wid])


def _make_gather():
    mesh = plsc.VectorSubcoreMesh(
        core_axis_name="c", subcore_axis_name="s",
        num_cores=NC, num_subcores=NS,
    )
    return pl.kernel(
        _gather_body,
        out_type=jax.ShapeDtypeStruct((NW, PER_W, D), jnp.float32),
        mesh=mesh,
        scratch_types=[
            pltpu.VMEM((PER_W,), jnp.int32),
            pltpu.VMEM((PER_W, D), jnp.float32),
            pltpu.SemaphoreType.DMA,
        ],
        compiler_params=pltpu.CompilerParams(needs_layout_passes=False),
    )


def _trans_body(x_ref, o_ref):
    parts = [x_ref[t].T.reshape(NF, 2, CHUNK) for t in range(NT)]
    o_ref[0] = jnp.concatenate(parts, axis=1)


def _transpose(r4):
    return pl.pallas_call(
        _trans_body,
        grid=(NB,),
        in_specs=[pl.BlockSpec((NT, CHUNK, D), lambda b: (b, 0, 0))],
        out_specs=pl.BlockSpec((1, NF, 2 * NT, CHUNK), lambda b: (b, 0, 0, 0)),
        out_shape=jax.ShapeDtypeStruct((NB, NF, 2 * NT, CHUNK), jnp.float32),
    )(r4)


def kernel(current_position_ids, past_position_ids, pos_emb_weight):
    w = _normalize(pos_emb_weight)
    gather = _make_gather()
    r_a = gather(w, current_position_ids).reshape(NG, CHUNK, D)
    r_b = gather(w, past_position_ids).reshape(NG, CHUNK, D)

    shp = current_position_ids.shape

    def assemble(p):
        f = p.reshape(NB, NF, NT, 2, CHUNK).transpose(0, 2, 4, 1, 3)
        return f.reshape(shp[0], shp[1], NF, 2)[:, None]

    return (assemble(_transpose(r_a)), assemble(_transpose(r_b)))
